# baseline (speedup 1.0000x reference)
import jax
import jax.numpy as jnp
from jax.experimental import pallas as pl
from jax.experimental.pallas import tpu as pltpu

_SB = 2048
_NBUF = 8


def _embed_body(t_hbm, w_hbm, o_hbm, tbuf, obuf, wbuf, in_sem, out_sem, w_sem):
    G, B, F = t_hbm.shape
    nper = B // _SB
    nch = G * nper

    def in_copy(c, slot):
        g = c // nper
        row = (c % nper) * _SB
        return pltpu.make_async_copy(
            t_hbm.at[g, pl.ds(row, _SB), :], tbuf.at[slot], in_sem.at[slot]
        )

    def out_copy(c, slot):
        g = c // nper
        row = (c % nper) * _SB
        return pltpu.make_async_copy(
            obuf.at[slot], o_hbm.at[g, :, pl.ds(row, _SB)], out_sem.at[slot]
        )

    for s in range(_NBUF):
        in_copy(s, s).start()
    w_copy = pltpu.make_async_copy(w_hbm, wbuf, w_sem)
    w_copy.start()
    w_copy.wait()

    def step(c, carry):
        slot = jax.lax.rem(c, _NBUF)
        in_copy(c, slot).wait()

        @pl.when(c >= _NBUF)
        def _():
            out_copy(c - _NBUF, slot).wait()

        g = c // nper
        obuf[slot] = jax.lax.dot_general(
            wbuf[g],
            tbuf[slot],
            dimension_numbers=(((0,), (1,)), ((), ())),
            preferred_element_type=jnp.float32,
        )
        out_copy(c, slot).start()

        @pl.when(c + _NBUF < nch)
        def _():
            in_copy(c + _NBUF, slot).start()

        return carry

    jax.lax.fori_loop(0, nch, step, 0)

    for s in range(_NBUF):
        c = nch - _NBUF + s
        out_copy(c, c % _NBUF).wait()


def kernel(tensor, W):
    G, B, F = tensor.shape
    E = W.shape[-1]
    out_t = pl.pallas_call(
        _embed_body,
        in_specs=[
            pl.BlockSpec(memory_space=pltpu.MemorySpace.HBM),
            pl.BlockSpec(memory_space=pltpu.MemorySpace.HBM),
        ],
        out_specs=pl.BlockSpec(memory_space=pltpu.MemorySpace.HBM),
        out_shape=jax.ShapeDtypeStruct((G, E, B), jnp.float32),
        scratch_shapes=[
            pltpu.VMEM((_NBUF, _SB, F), jnp.float32),
            pltpu.VMEM((_NBUF, E, _SB), jnp.float32),
            pltpu.VMEM((G, F, E), jnp.float32),
            pltpu.SemaphoreType.DMA((_NBUF,)),
            pltpu.SemaphoreType.DMA((_NBUF,)),
            pltpu.SemaphoreType.DMA,
        ],
    )(tensor, W)
    return jnp.swapaxes(out_t, 1, 2)


# speedup vs baseline: 1.0075x; 1.0075x over previous
import jax
import jax.numpy as jnp
from jax.experimental import pallas as pl
from jax.experimental.pallas import tpu as pltpu

_SB = 2048
_NBUF = 8


def _embed_body(t_hbm, w_ref, o_hbm, tbuf, obuf, in_sem, out_sem):
    G, B, F = t_hbm.shape
    E = w_ref.shape[-1]
    nper = B // _SB
    nch = G * nper
    rows = _SB // 8
    t4 = t_hbm.reshape(G, B // 8, 8, F)

    def in_copy(c, slot):
        g = c // nper
        prow = (c % nper) * rows
        return pltpu.make_async_copy(
            t4.at[g, pl.ds(prow, rows), :, :], tbuf.at[slot], in_sem.at[slot]
        )

    def out_copy(c, slot):
        g = c // nper
        row = (c % nper) * _SB
        return pltpu.make_async_copy(
            obuf.at[slot], o_hbm.at[g, :, pl.ds(row, _SB)], out_sem.at[slot]
        )

    for s in range(_NBUF):
        in_copy(s, s).start()

    def step(c, carry):
        slot = jax.lax.rem(c, _NBUF)
        in_copy(c, slot).wait()

        @pl.when(c >= _NBUF)
        def _():
            out_copy(c - _NBUF, slot).wait()

        g = c // nper
        x = tbuf[slot].reshape(_SB, F)
        obuf[slot] = jax.lax.dot_general(
            w_ref[g],
            x,
            dimension_numbers=(((0,), (1,)), ((), ())),
            preferred_element_type=jnp.float32,
        )
        out_copy(c, slot).start()

        @pl.when(c + _NBUF < nch)
        def _():
            in_copy(c + _NBUF, slot).start()

        return carry

    jax.lax.fori_loop(0, nch, step, 0)

    for s in range(_NBUF):
        c = nch - _NBUF + s
        out_copy(c, c % _NBUF).wait()


def kernel(tensor, W):
    G, B, F = tensor.shape
    E = W.shape[-1]
    rows = _SB // 8
    out_t = pl.pallas_call(
        _embed_body,
        in_specs=[
            pl.BlockSpec(memory_space=pltpu.MemorySpace.HBM),
            pl.BlockSpec(memory_space=pltpu.MemorySpace.VMEM),
        ],
        out_specs=pl.BlockSpec(memory_space=pltpu.MemorySpace.HBM),
        out_shape=jax.ShapeDtypeStruct((G, E, B), jnp.float32),
        scratch_shapes=[
            pltpu.VMEM((_NBUF, rows, 8, F), jnp.float32),
            pltpu.VMEM((_NBUF, E, _SB), jnp.float32),
            pltpu.SemaphoreType.DMA((_NBUF,)),
            pltpu.SemaphoreType.DMA((_NBUF,)),
        ],
    )(tensor, W)
    return jnp.swapaxes(out_t, 1, 2)


# speedup vs baseline: 1.0085x; 1.0010x over previous
import jax
import jax.numpy as jnp
from jax.experimental import pallas as pl
from jax.experimental.pallas import tpu as pltpu

_SB = 2048
_NBUF = 8
_NOB = 2


def _embed_body(t_hbm, w_ref, o_hbm, tbuf, obuf, in_sem, out_sem):
    G, B, F = t_hbm.shape
    E = w_ref.shape[-1]
    nper = B // _SB
    nch = G * nper
    rows = _SB // 8
    t4 = t_hbm.reshape(G, B // 8, 8, F)

    def in_copy(c, slot):
        g = c // nper
        prow = (c % nper) * rows
        return pltpu.make_async_copy(
            t4.at[g, pl.ds(prow, rows), :, :], tbuf.at[slot], in_sem.at[slot]
        )

    def out_copy(g, oslot):
        return pltpu.make_async_copy(
            obuf.at[oslot], o_hbm.at[g], out_sem.at[oslot]
        )

    for s in range(_NBUF):
        in_copy(s, s).start()

    def step(c, carry):
        slot = jax.lax.rem(c, _NBUF)
        g = c // nper
        i = jax.lax.rem(c, nper)
        oslot = jax.lax.rem(g, _NOB)
        in_copy(c, slot).wait()

        @pl.when(jnp.logical_and(i == 0, g >= _NOB))
        def _():
            out_copy(g - _NOB, oslot).wait()

        obuf[oslot, :, pl.ds(i * _SB, _SB)] = jax.lax.dot_general(
            w_ref[g],
            tbuf[slot].reshape(_SB, F),
            dimension_numbers=(((0,), (1,)), ((), ())),
            preferred_element_type=jnp.float32,
        )

        @pl.when(i == nper - 1)
        def _():
            out_copy(g, oslot).start()

        @pl.when(c + _NBUF < nch)
        def _():
            in_copy(c + _NBUF, slot).start()

        return carry

    jax.lax.fori_loop(0, nch, step, 0)

    for g in range(G - _NOB, G):
        out_copy(g, g % _NOB).wait()


def kernel(tensor, W):
    G, B, F = tensor.shape
    E = W.shape[-1]
    rows = _SB // 8
    out_t = pl.pallas_call(
        _embed_body,
        in_specs=[
            pl.BlockSpec(memory_space=pltpu.MemorySpace.HBM),
            pl.BlockSpec(memory_space=pltpu.MemorySpace.VMEM),
        ],
        out_specs=pl.BlockSpec(memory_space=pltpu.MemorySpace.HBM),
        out_shape=jax.ShapeDtypeStruct((G, E, B), jnp.float32),
        scratch_shapes=[
            pltpu.VMEM((_NBUF, rows, 8, F), jnp.float32),
            pltpu.VMEM((_NOB, E, B), jnp.float32),
            pltpu.SemaphoreType.DMA((_NBUF,)),
            pltpu.SemaphoreType.DMA((_NOB,)),
        ],
    )(tensor, W)
    return jnp.swapaxes(out_t, 1, 2)
